# baseline (speedup 1.0000x reference)
import functools

import jax
import jax.numpy as jnp
from jax import lax
from jax.experimental import pallas as pl
from jax.experimental.pallas import tpu as pltpu
from jax.experimental.pallas import tpu_sc as plsc

_NUM_CORES = 2
_NUM_SUBCORES = 16


def _sc_gather(table, idx):
    (B,) = idx.shape
    V, D = table.shape
    nw = _NUM_CORES * _NUM_SUBCORES
    b_per_w = B // nw

    def body(table_hbm, idx_hbm, out_hbm, idx_v, rows_v, sem):
        wid = lax.axis_index("s") * _NUM_CORES + lax.axis_index("c")
        base = wid * b_per_w
        pltpu.sync_copy(idx_hbm.at[pl.ds(base, b_per_w)], idx_v)
        pltpu.async_copy(table_hbm.at[idx_v], rows_v, sem).wait()
        pltpu.sync_copy(rows_v, out_hbm.at[pl.ds(base, b_per_w)])

    mesh = plsc.VectorSubcoreMesh(core_axis_name="c", subcore_axis_name="s")
    return pl.kernel(
        body,
        out_type=jax.ShapeDtypeStruct((B, D), jnp.float32),
        mesh=mesh,
        scratch_types=[
            pltpu.VMEM((b_per_w,), jnp.int32),
            pltpu.VMEM((b_per_w, D), jnp.float32),
            pltpu.SemaphoreType.DMA,
        ],
        compiler_params=pltpu.CompilerParams(use_tc_tiling_on_sc=False),
    )(table, idx)


def _tc_project(emb, w, bias_2d, tile_n=2048, nbuf=4):
    B, D = emb.shape
    V = w.shape[1]
    grid = pl.cdiv(V, tile_n)
    tail = V - (grid - 1) * tile_n

    def body(emb_ref, w_ref, b_ref, out_hbm, acc, tail_buf, sems, tail_sem):
        j = pl.program_id(0)
        nj = pl.num_programs(0)
        slot = jax.lax.rem(j, nbuf)

        @pl.when(j >= nbuf)
        def _():
            pltpu.make_async_copy(
                acc.at[slot], out_hbm.at[:, pl.ds(0, tile_n)], sems.at[slot]
            ).wait()

        val = jnp.broadcast_to(b_ref[...], (B, tile_n))

        @pl.when(j < nj - 1)
        def _():
            acc[slot] = val
            pltpu.make_async_copy(
                acc.at[slot],
                out_hbm.at[:, pl.ds(j * tile_n, tile_n)],
                sems.at[slot],
            ).start()

        @pl.when(j == nj - 1)
        def _():
            tail_buf[...] = val[:, :tail]
            tail_copy = pltpu.make_async_copy(
                tail_buf,
                out_hbm.at[:, pl.ds((grid - 1) * tile_n, tail)],
                tail_sem,
            )
            tail_copy.start()
            for d in range(1, min(nbuf, grid)):
                s = (grid - 1 - d) % nbuf
                pltpu.make_async_copy(
                    acc.at[s], out_hbm.at[:, pl.ds(0, tile_n)], sems.at[s]
                ).wait()
            tail_copy.wait()

    return pl.pallas_call(
        body,
        grid=(grid,),
        in_specs=[
            pl.BlockSpec((B, D), lambda j: (0, 0)),
            pl.BlockSpec((D, tile_n), lambda j: (0, j)),
            pl.BlockSpec((1, tile_n), lambda j: (0, j)),
        ],
        out_specs=pl.BlockSpec(memory_space=pl.ANY),
        out_shape=jax.ShapeDtypeStruct((B, V), jnp.float32),
        scratch_shapes=[
            pltpu.VMEM((nbuf, B, tile_n), jnp.float32),
            pltpu.VMEM((B, tail), jnp.float32),
            pltpu.SemaphoreType.DMA((nbuf,)),
            pltpu.SemaphoreType.DMA,
        ],
    )(emb, w, bias_2d)


def _tc_project_rows(emb, w, bias_2d, tile_m=32, ndma=4):
    B, D = emb.shape
    V = w.shape[1]
    grid = B // tile_m
    rows_per_dma = tile_m // ndma

    def body(emb_ref, w_ref, b_ref, out_hbm, buf, sems):
        j = pl.program_id(0)
        p = jax.lax.rem(j, 2)

        @pl.when(j >= 2)
        def _():
            for k in range(ndma):
                pltpu.make_async_copy(
                    buf.at[p, pl.ds(k * rows_per_dma, rows_per_dma)],
                    out_hbm.at[pl.ds(0, rows_per_dma)],
                    sems.at[p, k],
                ).wait()

        buf[p] = (
            jnp.dot(emb_ref[...], w_ref[...], preferred_element_type=jnp.float32)
            + b_ref[...]
        )
        base = j * tile_m
        for k in range(ndma):
            pltpu.make_async_copy(
                buf.at[p, pl.ds(k * rows_per_dma, rows_per_dma)],
                out_hbm.at[pl.ds(base + k * rows_per_dma, rows_per_dma)],
                sems.at[p, k],
            ).start()

        @pl.when(j == grid - 1)
        def _():
            for q in range(2):
                for k in range(ndma):
                    pltpu.make_async_copy(
                        buf.at[q, pl.ds(k * rows_per_dma, rows_per_dma)],
                        out_hbm.at[pl.ds(0, rows_per_dma)],
                        sems.at[q, k],
                    ).wait()

    return pl.pallas_call(
        body,
        grid=(grid,),
        in_specs=[
            pl.BlockSpec((tile_m, D), lambda i: (i, 0)),
            pl.BlockSpec((D, V), lambda i: (0, 0)),
            pl.BlockSpec((1, V), lambda i: (0, 0)),
        ],
        out_specs=pl.BlockSpec(memory_space=pl.ANY),
        out_shape=jax.ShapeDtypeStruct((B, V), jnp.float32),
        scratch_shapes=[
            pltpu.VMEM((2, tile_m, V), jnp.float32),
            pltpu.SemaphoreType.DMA((2, ndma)),
        ],
        compiler_params=pltpu.CompilerParams(
            vmem_limit_bytes=56 * 1024 * 1024,
        ),
    )(emb, w, bias_2d)


def kernel(input_data, embedding_table, fc_weight, fc_bias):
    emb = _sc_gather(embedding_table, input_data.astype(jnp.int32))
    return _tc_project_rows(
        emb.astype(jnp.bfloat16),
        fc_weight.astype(jnp.bfloat16),
        fc_bias.reshape(1, -1),
    )


# speedup vs baseline: 1.9758x; 1.9758x over previous
import jax
import jax.numpy as jnp
from jax import lax
from jax.experimental import pallas as pl
from jax.experimental.pallas import tpu as pltpu
from jax.experimental.pallas import tpu_sc as plsc

_NUM_CORES = 2
_NUM_SUBCORES = 16


def _sc_gather(table, idx):
    (B,) = idx.shape
    V, D = table.shape
    nw = _NUM_CORES * _NUM_SUBCORES
    b_per_w = B // nw

    def body(table_hbm, idx_hbm, out_hbm, idx_v, rows_v, sem):
        wid = lax.axis_index("s") * _NUM_CORES + lax.axis_index("c")
        base = wid * b_per_w
        pltpu.sync_copy(idx_hbm.at[pl.ds(base, b_per_w)], idx_v)
        pltpu.async_copy(table_hbm.at[idx_v], rows_v, sem).wait()
        pltpu.sync_copy(rows_v, out_hbm.at[pl.ds(base, b_per_w)])

    mesh = plsc.VectorSubcoreMesh(core_axis_name="c", subcore_axis_name="s")
    return pl.kernel(
        body,
        out_type=jax.ShapeDtypeStruct((B, D), jnp.float32),
        mesh=mesh,
        scratch_types=[
            pltpu.VMEM((b_per_w,), jnp.int32),
            pltpu.VMEM((b_per_w, D), jnp.float32),
            pltpu.SemaphoreType.DMA,
        ],
        compiler_params=pltpu.CompilerParams(use_tc_tiling_on_sc=False),
    )(table, idx)


def _mm_body(wt_ref, embt_ref, b_ref, out_ref):
    out_ref[...] = (
        jnp.dot(wt_ref[...], embt_ref[...], preferred_element_type=jnp.float32)
        + b_ref[...]
    )


def _tc_project_t(wt, embt, bias_col, tile_v=2000):
    V, D = wt.shape
    B = embt.shape[1]
    return pl.pallas_call(
        _mm_body,
        grid=(V // tile_v,),
        in_specs=[
            pl.BlockSpec((tile_v, D), lambda j: (j, 0)),
            pl.BlockSpec((D, B), lambda j: (0, 0)),
            pl.BlockSpec((tile_v, 1), lambda j: (j, 0)),
        ],
        out_specs=pl.BlockSpec((tile_v, B), lambda j: (j, 0)),
        out_shape=jax.ShapeDtypeStruct((V, B), jnp.float32),
    )(wt, embt, bias_col)


def kernel(input_data, embedding_table, fc_weight, fc_bias):
    emb = _sc_gather(embedding_table, input_data.astype(jnp.int32))
    out_t = _tc_project_t(
        fc_weight.T.astype(jnp.bfloat16),
        emb.T.astype(jnp.bfloat16),
        fc_bias.reshape(-1, 1),
    )
    return out_t.T


# speedup vs baseline: 2.5106x; 1.2707x over previous
import jax
import jax.numpy as jnp
from jax import lax
from jax.experimental import pallas as pl
from jax.experimental.pallas import tpu as pltpu
from jax.experimental.pallas import tpu_sc as plsc

_NUM_CORES = 2
_NUM_SUBCORES = 16


def _sc_gather(table, idx):
    (B,) = idx.shape
    V, D = table.shape
    nw = _NUM_CORES * _NUM_SUBCORES
    b_per_w = B // nw

    def body(table_hbm, idx_hbm, out_hbm, idx_v, rows_v, sem):
        wid = lax.axis_index("s") * _NUM_CORES + lax.axis_index("c")
        base = wid * b_per_w
        pltpu.sync_copy(idx_hbm.at[pl.ds(base, b_per_w)], idx_v)
        pltpu.async_copy(table_hbm.at[idx_v], rows_v, sem).wait()
        pltpu.sync_copy(rows_v, out_hbm.at[pl.ds(base, b_per_w)])

    mesh = plsc.VectorSubcoreMesh(core_axis_name="c", subcore_axis_name="s")
    return pl.kernel(
        body,
        out_type=jax.ShapeDtypeStruct((B, D), jnp.float32),
        mesh=mesh,
        scratch_types=[
            pltpu.VMEM((b_per_w,), jnp.int32),
            pltpu.VMEM((b_per_w, D), jnp.float32),
            pltpu.SemaphoreType.DMA,
        ],
        compiler_params=pltpu.CompilerParams(use_tc_tiling_on_sc=False),
    )(table, idx)


def _mm_body(wt_ref, embt_ref, out_ref):
    out_ref[...] = jnp.dot(
        wt_ref[...], embt_ref[...], preferred_element_type=jnp.float32
    )


def _tc_project_t(wt, embt, tile_v=2000):
    V, D = wt.shape
    B = embt.shape[1]
    return pl.pallas_call(
        _mm_body,
        grid=(V // tile_v,),
        in_specs=[
            pl.BlockSpec((tile_v, D), lambda j: (j, 0)),
            pl.BlockSpec((D, B), lambda j: (0, 0)),
        ],
        out_specs=pl.BlockSpec((tile_v, B), lambda j: (j, 0)),
        out_shape=jax.ShapeDtypeStruct((V, B), jnp.float32),
    )(wt, embt)


def kernel(input_data, embedding_table, fc_weight, fc_bias):
    emb = _sc_gather(embedding_table, input_data.astype(jnp.int32))
    B = emb.shape[0]
    w_aug = jnp.concatenate([fc_weight, fc_bias[None, :]], axis=0)
    embt_aug = jnp.concatenate(
        [emb.T, jnp.ones((1, B), jnp.float32)], axis=0
    )
    out_t = _tc_project_t(
        w_aug.T.astype(jnp.bfloat16),
        embt_aug.astype(jnp.bfloat16),
    )
    return out_t.T


# speedup vs baseline: 2.5267x; 1.0064x over previous
import jax
import jax.numpy as jnp
from jax import lax
from jax.experimental import pallas as pl
from jax.experimental.pallas import tpu as pltpu
from jax.experimental.pallas import tpu_sc as plsc

_NUM_CORES = 2
_NUM_SUBCORES = 16


def _sc_gather(table, idx):
    (B,) = idx.shape
    V, D = table.shape
    nw = _NUM_CORES * _NUM_SUBCORES
    b_per_w = B // nw

    def body(table_hbm, idx_hbm, out_hbm, idx_v, rows_v, sem):
        wid = lax.axis_index("s") * _NUM_CORES + lax.axis_index("c")
        base = wid * b_per_w
        pltpu.sync_copy(idx_hbm.at[pl.ds(base, b_per_w)], idx_v)
        pltpu.async_copy(table_hbm.at[idx_v], rows_v, sem).wait()
        pltpu.sync_copy(rows_v, out_hbm.at[pl.ds(base, b_per_w)])

    mesh = plsc.VectorSubcoreMesh(core_axis_name="c", subcore_axis_name="s")
    return pl.kernel(
        body,
        out_type=jax.ShapeDtypeStruct((B, D), jnp.float32),
        mesh=mesh,
        scratch_types=[
            pltpu.VMEM((b_per_w,), jnp.int32),
            pltpu.VMEM((b_per_w, D), jnp.float32),
            pltpu.SemaphoreType.DMA,
        ],
        compiler_params=pltpu.CompilerParams(use_tc_tiling_on_sc=False),
    )(table, idx)


def _mm_body(wt_ref, embt_ref, out_ref):
    out_ref[...] = jnp.dot(
        wt_ref[...], embt_ref[...], preferred_element_type=jnp.float32
    )


def _tc_project_t(wt, embt, tile_v=4000):
    V, D = wt.shape
    B = embt.shape[1]
    return pl.pallas_call(
        _mm_body,
        grid=(V // tile_v,),
        in_specs=[
            pl.BlockSpec((tile_v, D), lambda j: (j, 0)),
            pl.BlockSpec((D, B), lambda j: (0, 0)),
        ],
        out_specs=pl.BlockSpec((tile_v, B), lambda j: (j, 0)),
        out_shape=jax.ShapeDtypeStruct((V, B), jnp.float32),
    )(wt, embt)


def kernel(input_data, embedding_table, fc_weight, fc_bias):
    emb = _sc_gather(embedding_table, input_data.astype(jnp.int32))
    B = emb.shape[0]
    w_aug = jnp.concatenate([fc_weight, fc_bias[None, :]], axis=0)
    embt_aug = jnp.concatenate(
        [emb.T, jnp.ones((1, B), jnp.float32)], axis=0
    )
    out_t = _tc_project_t(
        w_aug.T.astype(jnp.bfloat16),
        embt_aug.astype(jnp.bfloat16),
    )
    return out_t.T


# speedup vs baseline: 2.9767x; 1.1781x over previous
import jax
import jax.numpy as jnp
from jax import lax
from jax.experimental import pallas as pl
from jax.experimental.pallas import tpu as pltpu
from jax.experimental.pallas import tpu_sc as plsc

_NUM_CORES = 2
_NUM_SUBCORES = 16


def _sc_gather(table, idx):
    (B,) = idx.shape
    V, D = table.shape
    nw = _NUM_CORES * _NUM_SUBCORES
    b_per_w = B // nw
    g_per_w = b_per_w // 8

    table3 = table.reshape(V // 8, 8, D)

    def body(table_hbm, idx_hbm, out_hbm, idx_v, rows_v, out_v, sem):
        wid = lax.axis_index("s") * _NUM_CORES + lax.axis_index("c")
        base = wid * b_per_w
        pltpu.sync_copy(idx_hbm.at[pl.ds(base, b_per_w)], idx_v)
        iota = lax.iota(jnp.int32, 16)
        for g in range(b_per_w // 16):
            tchunk = lax.shift_right_logical(idx_v[pl.ds(g * 16, 16)], 3)
            for l in range(16):
                b = g * 16 + l
                tid = jnp.max(jnp.where(iota == l, tchunk, 0))
                pltpu.make_async_copy(
                    table_hbm.at[tid], rows_v.at[b], sem
                ).start()
        for b in range(b_per_w):
            pltpu.make_async_copy(table_hbm.at[0], rows_v.at[0], sem).wait()
        for g in range(b_per_w // 16):
            bvec = iota + g * 16
            svec = lax.rem(idx_v[pl.ds(g * 16, 16)], 8)
            for d in range(D):
                dfull = jnp.full((16,), d, jnp.int32)
                vals = plsc.load_gather(rows_v, [bvec, svec, dfull])
                plsc.store_scatter(
                    out_v,
                    [lax.shift_right_logical(bvec, 3), lax.rem(bvec, 8), dfull],
                    vals,
                )
        pltpu.sync_copy(out_v, out_hbm.at[pl.ds(wid * g_per_w, g_per_w)])

    mesh = plsc.VectorSubcoreMesh(core_axis_name="c", subcore_axis_name="s")
    out3 = pl.kernel(
        body,
        out_type=jax.ShapeDtypeStruct((B // 8, 8, D), jnp.float32),
        mesh=mesh,
        scratch_types=[
            pltpu.VMEM((b_per_w,), jnp.int32),
            pltpu.VMEM((b_per_w, 8, D), jnp.float32),
            pltpu.VMEM((g_per_w, 8, D), jnp.float32),
            pltpu.SemaphoreType.DMA,
        ],
        compiler_params=pltpu.CompilerParams(needs_layout_passes=False),
    )(table3, idx)
    return out3.reshape(B, D)


def _mm_body(wt_ref, embt_ref, out_ref):
    out_ref[...] = jnp.dot(
        wt_ref[...], embt_ref[...], preferred_element_type=jnp.float32
    )


def _tc_project_t(wt, embt, tile_v=4000):
    V, D = wt.shape
    B = embt.shape[1]
    return pl.pallas_call(
        _mm_body,
        grid=(V // tile_v,),
        in_specs=[
            pl.BlockSpec((tile_v, D), lambda j: (j, 0)),
            pl.BlockSpec((D, B), lambda j: (0, 0)),
        ],
        out_specs=pl.BlockSpec((tile_v, B), lambda j: (j, 0)),
        out_shape=jax.ShapeDtypeStruct((V, B), jnp.float32),
    )(wt, embt)


def kernel(input_data, embedding_table, fc_weight, fc_bias):
    emb = _sc_gather(embedding_table, input_data.astype(jnp.int32))
    B = emb.shape[0]
    w_aug = jnp.concatenate([fc_weight, fc_bias[None, :]], axis=0)
    embt_aug = jnp.concatenate(
        [emb.T, jnp.ones((1, B), jnp.float32)], axis=0
    )
    out_t = _tc_project_t(
        w_aug.T.astype(jnp.bfloat16),
        embt_aug.astype(jnp.bfloat16),
    )
    return out_t.T


# speedup vs baseline: 3.1924x; 1.0725x over previous
import jax
import jax.numpy as jnp
from jax import lax
from jax.experimental import pallas as pl
from jax.experimental.pallas import tpu as pltpu
from jax.experimental.pallas import tpu_sc as plsc

_NUM_CORES = 2
_NUM_SUBCORES = 16


def _sc_gather(table, idx):
    (B,) = idx.shape
    V, D = table.shape
    nw = _NUM_CORES * _NUM_SUBCORES
    b_per_w = B // nw
    g_per_w = b_per_w // 8

    table_t = table.T

    def body(table_hbm, idx_hbm, out_hbm, idx_v, off_v, rows_v, out_v, sem):
        wid = lax.axis_index("s") * _NUM_CORES + lax.axis_index("c")
        base = wid * b_per_w
        pltpu.sync_copy(idx_hbm.at[pl.ds(base, b_per_w)], idx_v)
        iota = lax.iota(jnp.int32, 16)
        for g in range(b_per_w // 16):
            sl = pl.ds(g * 16, 16)
            off_v[sl] = jax.lax.bitwise_and(idx_v[sl], jnp.int32(~127))
        for g in range(b_per_w // 16):
            ochunk = off_v[pl.ds(g * 16, 16)]
            for l in range(16):
                b = g * 16 + l
                off = pl.multiple_of(jnp.max(jnp.where(iota == l, ochunk, 0)), 128)
                pltpu.make_async_copy(
                    table_hbm.at[:, pl.ds(off, 128)], rows_v.at[b], sem
                ).start()
        for b in range(b_per_w):
            pltpu.make_async_copy(
                table_hbm.at[:, pl.ds(0, 128)], rows_v.at[0], sem
            ).wait()
        for g in range(b_per_w // 16):
            bvec = iota + g * 16
            sl = pl.ds(g * 16, 16)
            lvec = idx_v[sl] - off_v[sl]
            for d in range(D):
                dfull = jnp.full((16,), d, jnp.int32)
                vals = plsc.load_gather(rows_v, [bvec, dfull, lvec])
                plsc.store_scatter(
                    out_v,
                    [lax.shift_right_logical(bvec, 3), lax.rem(bvec, 8), dfull],
                    vals,
                )
        pltpu.sync_copy(out_v, out_hbm.at[pl.ds(wid * g_per_w, g_per_w)])

    mesh = plsc.VectorSubcoreMesh(core_axis_name="c", subcore_axis_name="s")
    out3 = pl.kernel(
        body,
        out_type=jax.ShapeDtypeStruct((B // 8, 8, D), jnp.float32),
        mesh=mesh,
        scratch_types=[
            pltpu.VMEM((b_per_w,), jnp.int32),
            pltpu.VMEM((b_per_w,), jnp.int32),
            pltpu.VMEM((b_per_w, D, 128), jnp.float32),
            pltpu.VMEM((g_per_w, 8, D), jnp.float32),
            pltpu.SemaphoreType.DMA,
        ],
        compiler_params=pltpu.CompilerParams(
            needs_layout_passes=False, disable_bounds_checks=True
        ),
    )(table_t, idx)
    return out3.reshape(B, D)


def _mm_body(wt_ref, embt_ref, out_ref):
    out_ref[...] = jnp.dot(
        wt_ref[...], embt_ref[...], preferred_element_type=jnp.float32
    )


def _tc_project_t(wt, embt, tile_v=5000):
    V, D = wt.shape
    B = embt.shape[1]
    return pl.pallas_call(
        _mm_body,
        grid=(V // tile_v,),
        in_specs=[
            pl.BlockSpec((tile_v, D), lambda j: (j, 0)),
            pl.BlockSpec((D, B), lambda j: (0, 0)),
        ],
        out_specs=pl.BlockSpec((tile_v, B), lambda j: (j, 0)),
        out_shape=jax.ShapeDtypeStruct((V, B), jnp.float32),
    )(wt, embt)


def kernel(input_data, embedding_table, fc_weight, fc_bias):
    emb = _sc_gather(embedding_table, input_data.astype(jnp.int32))
    B = emb.shape[0]
    w_aug = jnp.concatenate([fc_weight, fc_bias[None, :]], axis=0)
    embt_aug = jnp.concatenate(
        [emb.T, jnp.ones((1, B), jnp.float32)], axis=0
    )
    out_t = _tc_project_t(
        w_aug.T.astype(jnp.bfloat16),
        embt_aug.astype(jnp.bfloat16),
    )
    return out_t.T
